# baseline (speedup 1.0000x reference)
import jax
import jax.numpy as jnp
from jax import lax
from jax.experimental import pallas as pl
from jax.experimental.pallas import tpu as pltpu
from jax.experimental.pallas import tpu_sc as plsc

N = 10000
E = 320000
D = 128

NC = 2
NS = 16
L = 16
NW = NC * NS
EW = E // NW
K = 80
T = EW // K
NP = 10240
RPT = NP // NS
DG = D // L


def _sc_body(xb_hbm, ei_hbm, ev_hbm, out_hbm,
             src_v, rows_0, rows_1, rows_2, outf_0, outf_1,
             dib_0, dib_1, dib_2, evb_0, evb_1, evb_2, y_sp,
             gsem_0, gsem_1, gsem_2, ssem_0, ssem_1, ssem_2):
    c = lax.axis_index("c")
    s = lax.axis_index("s")
    wid = s * NC + c
    ebase = wid * EW
    sbase = E + ebase

    rows = (rows_0, rows_1, rows_2)
    outf = (outf_0, outf_1)
    dib = (dib_0, dib_1, dib_2)
    evb = (evb_0, evb_1, evb_2)
    gsem = (gsem_0, gsem_1, gsem_2)
    ssem = (ssem_0, ssem_1, ssem_2)

    zvec = jnp.zeros((L,), jnp.float32)

    @pl.loop(0, K)
    def _zfill(r):
        for g in range(DG):
            outf_0[r, pl.ds(L * g, L)] = zvec

    row0 = s * RPT

    @pl.loop(0, RPT // K)
    def _zcopy(j):
        pltpu.sync_copy(outf_0, y_sp.at[pl.ds(row0 + j * K, K)])

    pltpu.sync_copy(ei_hbm.at[pl.ds(sbase, EW)], src_v)

    plsc.subcore_barrier()

    def fetch(t, b):
        pltpu.async_copy(ei_hbm.at[pl.ds(ebase + t * K, K)], dib[b], gsem[b])
        pltpu.async_copy(ev_hbm.at[pl.ds(ebase + t * K, K)], evb[b], gsem[b])
        pltpu.async_copy(xb_hbm.at[src_v.at[pl.ds(t * K, K)]], rows[b], gsem[b])

    def wait_f(b):
        pltpu.make_async_copy(ei_hbm.at[pl.ds(0, K)], dib[b], gsem[b]).wait()
        pltpu.make_async_copy(ev_hbm.at[pl.ds(0, K)], evb[b], gsem[b]).wait()
        pltpu.make_async_copy(xb_hbm.at[pl.ds(0, K)], rows[b], gsem[b]).wait()

    def scatter(b, p):
        pltpu.async_copy(outf[p], y_sp.at[dib[b]], ssem[b], add=True)

    def wait_s(b):
        pltpu.make_async_copy(outf_0, y_sp.at[pl.ds(0, K)], ssem[b]).wait()

    def scale(b, p):
        @pl.loop(0, K // L)
        def _egroup(eb):
            ev16 = evb[b][pl.ds(eb * L, L)]
            for j in range(L):
                evv = jnp.full((L,), ev16[j], jnp.float32)
                e = eb * L + j
                for g in range(DG // 2):
                    w = rows[b][e, pl.ds(L * g, L)]
                    va = plsc.bitcast(lax.shift_left(w, 16), jnp.float32)
                    vb = plsc.bitcast(w & jnp.int32(-65536), jnp.float32)
                    outf[p][e, pl.ds(2 * L * g, L)] = va * evv
                    outf[p][e, pl.ds(2 * L * g + L, L)] = vb * evv

    fetch(0, 0)
    fetch(1, 1)

    @pl.loop(0, T // 6)
    def _six(g):
        for k in range(6):
            t = 6 * g + k
            b = k % 3
            b2 = (k + 2) % 3
            p = k % 2
            wait_f(b)
            if k == 0:
                @pl.when(g > 0)
                def _():
                    wait_s(b2)
            else:
                wait_s(b2)
            fetch(t + 2, b2)
            scale(b, p)
            scatter(b, p)

    for k in range(5):
        t = T - 5 + k
        b = k % 3
        b2 = (k + 2) % 3
        p = k % 2
        wait_f(b)
        wait_s(b2)
        if k < 3:
            fetch(t + 2, b2)
        scale(b, p)
        scatter(b, p)
    wait_s(1)

    plsc.subcore_barrier()

    obase = c * NP + s * RPT

    @pl.loop(0, RPT // K)
    def _flush(j):
        pltpu.sync_copy(y_sp.at[pl.ds(row0 + j * K, K)], outf_0)
        pltpu.sync_copy(outf_0, out_hbm.at[pl.ds(obase + j * K, K)])


@jax.jit
def _sc_scatter(xb, ei_flat, ev):
    mesh = plsc.VectorSubcoreMesh(core_axis_name="c", subcore_axis_name="s")
    f = pl.kernel(
        _sc_body,
        out_type=jax.ShapeDtypeStruct((2 * NP, D), jnp.float32),
        mesh=mesh,
        compiler_params=pltpu.CompilerParams(needs_layout_passes=False, use_tc_tiling_on_sc=False),
        scratch_types=[
            pltpu.VMEM((EW,), jnp.int32),
            pltpu.VMEM((K, D // 2), jnp.int32),
            pltpu.VMEM((K, D // 2), jnp.int32),
            pltpu.VMEM((K, D // 2), jnp.int32),
            pltpu.VMEM((K, D), jnp.float32),
            pltpu.VMEM((K, D), jnp.float32),
            pltpu.VMEM((K,), jnp.int32),
            pltpu.VMEM((K,), jnp.int32),
            pltpu.VMEM((K,), jnp.int32),
            pltpu.VMEM((K,), jnp.float32),
            pltpu.VMEM((K,), jnp.float32),
            pltpu.VMEM((K,), jnp.float32),
            pltpu.VMEM_SHARED((NP, D), jnp.float32),
            pltpu.SemaphoreType.DMA,
            pltpu.SemaphoreType.DMA,
            pltpu.SemaphoreType.DMA,
            pltpu.SemaphoreType.DMA,
            pltpu.SemaphoreType.DMA,
            pltpu.SemaphoreType.DMA,
        ],
    )
    return f(xb, ei_flat, ev)


def _tc_body(yp_ref, x_ref, w0_ref, w1_ref, wt_ref, g0_ref, b0_ref,
             g1_ref, b1_ref, nn_ref, out_ref):
    eps = 1e-5
    x = x_ref[...]
    y = yp_ref[0:N, :] + yp_ref[NP:NP + N, :] + wt_ref[0, 0] * x
    v = lax.dot_general(y, w0_ref[...], (((1,), (1,)), ((), ())),
                        preferred_element_type=jnp.float32)
    m0 = jnp.mean(v, axis=0, keepdims=True)
    d0 = v - m0
    var0 = jnp.mean(d0 * d0, axis=0, keepdims=True)
    h = jnp.maximum(d0 * (g0_ref[...] * lax.rsqrt(var0 + eps)) + b0_ref[...],
                    0.0)
    u = lax.dot_general(h, w1_ref[...], (((1,), (1,)), ((), ())),
                        preferred_element_type=jnp.float32)
    u = u * nn_ref[...]
    m1 = jnp.mean(u, axis=0, keepdims=True)
    d1 = u - m1
    var1 = jnp.mean(d1 * d1, axis=0, keepdims=True)
    out = jnp.maximum(d1 * (g1_ref[...] * lax.rsqrt(var1 + eps)) + b1_ref[...],
                      0.0)
    out_ref[...] = out + x


@jax.jit
def _tc_epilogue(yp, x, W0, W1, weight, g0, b0, g1, b1, n_norm):
    return pl.pallas_call(
        _tc_body,
        out_shape=jax.ShapeDtypeStruct((N, D), jnp.float32),
    )(yp, x, W0, W1, weight.reshape(1, 1), g0.reshape(1, D),
      b0.reshape(1, D), g1.reshape(1, D), b1.reshape(1, D), n_norm)


def kernel(x, edge_index, edge_values, n_norm, W0, W1, weight, g0, b0, g1, b1):
    xb = x.reshape(N, 4, 2, 16).transpose(0, 1, 3, 2).reshape(N, D)
    xb = xb.astype(jnp.bfloat16)
    xq = lax.bitcast_convert_type(xb.reshape(N, D // 2, 2), jnp.int32)
    yp = _sc_scatter(xq, edge_index.reshape(2 * E), edge_values)
    return _tc_epilogue(yp, x, W0, W1, weight, g0, b0, g1, b1, n_norm)


# speedup vs baseline: 2.2893x; 2.2893x over previous
import jax
import jax.numpy as jnp
from jax import lax
from jax.experimental import pallas as pl
from jax.experimental.pallas import tpu as pltpu
from jax.experimental.pallas import tpu_sc as plsc

N = 10000
E = 320000
D = 128

NC = 2
NS = 16
L = 16
NW = NC * NS
EW = E // NW
K = 80
T = EW // K
NP = 10240
RPT = NP // NS
DG = D // L


def _sc_body(x_hbm, ei_hbm, ev_hbm, out_hbm,
             src_v, rows_0, rows_1, rows_2, dib_0, dib_1, dib_2,
             evb_0, evb_1, evb_2, y_sp,
             gsem_0, gsem_1, gsem_2, ssem_0, ssem_1, ssem_2):
    c = lax.axis_index("c")
    s = lax.axis_index("s")
    wid = s * NC + c
    ebase = wid * EW
    sbase = E + ebase

    rows = (rows_0, rows_1, rows_2)
    dib = (dib_0, dib_1, dib_2)
    evb = (evb_0, evb_1, evb_2)
    gsem = (gsem_0, gsem_1, gsem_2)
    ssem = (ssem_0, ssem_1, ssem_2)

    zvec = jnp.zeros((L,), jnp.float32)

    @pl.loop(0, K)
    def _zfill(r):
        for g in range(DG):
            rows_0[r, pl.ds(L * g, L)] = zvec

    row0 = s * RPT

    @pl.loop(0, RPT // K)
    def _zcopy(j):
        pltpu.sync_copy(rows_0, y_sp.at[pl.ds(row0 + j * K, K)])

    pltpu.sync_copy(ei_hbm.at[pl.ds(sbase, EW)], src_v)

    plsc.subcore_barrier()

    def fetch(t, b):
        pltpu.async_copy(ei_hbm.at[pl.ds(ebase + t * K, K)], dib[b], gsem[b])
        pltpu.async_copy(ev_hbm.at[pl.ds(ebase + t * K, K)], evb[b], gsem[b])
        pltpu.async_copy(x_hbm.at[src_v.at[pl.ds(t * K, K)]], rows[b], gsem[b])

    def wait_f(b):
        pltpu.make_async_copy(ei_hbm.at[pl.ds(0, K)], dib[b], gsem[b]).wait()
        pltpu.make_async_copy(ev_hbm.at[pl.ds(0, K)], evb[b], gsem[b]).wait()
        pltpu.make_async_copy(x_hbm.at[pl.ds(0, K)], rows[b], gsem[b]).wait()

    def scatter(b):
        pltpu.async_copy(rows[b], y_sp.at[dib[b]], ssem[b], add=True)

    def wait_s(b):
        pltpu.make_async_copy(rows[b], y_sp.at[pl.ds(0, K)], ssem[b]).wait()

    def scale(b):
        @pl.loop(0, K // L)
        def _egroup(eb):
            ev16 = evb[b][pl.ds(eb * L, L)]
            for j in range(L):
                evv = jnp.full((L,), ev16[j], jnp.float32)
                e = eb * L + j
                for g in range(DG):
                    sl = pl.ds(L * g, L)
                    rows[b][e, sl] = rows[b][e, sl] * evv

    fetch(0, 0)
    fetch(1, 1)

    @pl.loop(0, (T - 2) // 3)
    def _trip(g):
        for k in range(3):
            t = 3 * g + k
            b = k
            b2 = (k + 2) % 3
            wait_f(b)
            scale(b)
            if k == 0:
                @pl.when(g > 0)
                def _():
                    wait_s(b2)
            else:
                wait_s(b2)
            fetch(t + 2, b2)
            scatter(b)

    wait_f(0)
    wait_s(2)
    scale(0)
    scatter(0)
    wait_f(1)
    wait_s(0)
    scale(1)
    scatter(1)
    wait_s(1)

    plsc.subcore_barrier()

    obase = c * NP + s * RPT

    @pl.loop(0, RPT // K)
    def _flush(j):
        pltpu.sync_copy(y_sp.at[pl.ds(row0 + j * K, K)], rows_0)
        pltpu.sync_copy(rows_0, out_hbm.at[pl.ds(obase + j * K, K)])


@jax.jit
def _sc_scatter(x, ei_flat, ev):
    mesh = plsc.VectorSubcoreMesh(core_axis_name="c", subcore_axis_name="s")
    f = pl.kernel(
        _sc_body,
        out_type=jax.ShapeDtypeStruct((2 * NP, D), jnp.float32),
        mesh=mesh,
        scratch_types=[
            pltpu.VMEM((EW,), jnp.int32),
            pltpu.VMEM((K, D), jnp.float32),
            pltpu.VMEM((K, D), jnp.float32),
            pltpu.VMEM((K, D), jnp.float32),
            pltpu.VMEM((K,), jnp.int32),
            pltpu.VMEM((K,), jnp.int32),
            pltpu.VMEM((K,), jnp.int32),
            pltpu.VMEM((K,), jnp.float32),
            pltpu.VMEM((K,), jnp.float32),
            pltpu.VMEM((K,), jnp.float32),
            pltpu.VMEM_SHARED((NP, D), jnp.float32),
            pltpu.SemaphoreType.DMA,
            pltpu.SemaphoreType.DMA,
            pltpu.SemaphoreType.DMA,
            pltpu.SemaphoreType.DMA,
            pltpu.SemaphoreType.DMA,
            pltpu.SemaphoreType.DMA,
        ],
    )
    return f(x, ei_flat, ev)


def _tc_body(yp_ref, x_ref, w0_ref, w1_ref, wt_ref, g0_ref, b0_ref,
             g1_ref, b1_ref, nn_ref, out_ref):
    eps = 1e-5
    x = x_ref[...]
    y = yp_ref[0:N, :] + yp_ref[NP:NP + N, :] + wt_ref[0, 0] * x
    v = lax.dot_general(y, w0_ref[...], (((1,), (1,)), ((), ())),
                        preferred_element_type=jnp.float32)
    m0 = jnp.mean(v, axis=0, keepdims=True)
    d0 = v - m0
    var0 = jnp.mean(d0 * d0, axis=0, keepdims=True)
    h = jnp.maximum(d0 * (g0_ref[...] * lax.rsqrt(var0 + eps)) + b0_ref[...],
                    0.0)
    u = lax.dot_general(h, w1_ref[...], (((1,), (1,)), ((), ())),
                        preferred_element_type=jnp.float32)
    u = u * nn_ref[...]
    m1 = jnp.mean(u, axis=0, keepdims=True)
    d1 = u - m1
    var1 = jnp.mean(d1 * d1, axis=0, keepdims=True)
    out = jnp.maximum(d1 * (g1_ref[...] * lax.rsqrt(var1 + eps)) + b1_ref[...],
                      0.0)
    out_ref[...] = out + x


@jax.jit
def _tc_epilogue(yp, x, W0, W1, weight, g0, b0, g1, b1, n_norm):
    return pl.pallas_call(
        _tc_body,
        out_shape=jax.ShapeDtypeStruct((N, D), jnp.float32),
    )(yp, x, W0, W1, weight.reshape(1, 1), g0.reshape(1, D),
      b0.reshape(1, D), g1.reshape(1, D), b1.reshape(1, D), n_norm)


def kernel(x, edge_index, edge_values, n_norm, W0, W1, weight, g0, b0, g1, b1):
    yp = _sc_scatter(x, edge_index.reshape(2 * E), edge_values)
    return _tc_epilogue(yp, x, W0, W1, weight, g0, b0, g1, b1, n_norm)
